# baseline (speedup 1.0000x reference)
import functools

import jax
import jax.numpy as jnp
from jax.experimental import pallas as pl
from jax.experimental.pallas import tpu as pltpu


def _conv_one(xb, w_ref, b_ref, h):
    c, hw = xb.shape
    w = hw // h
    ho, wo = h // 2, w // 2
    m = ho * wo
    cout = w_ref.shape[2]

    xt = jnp.transpose(xb)
    xi = pltpu.bitcast(xt, jnp.int32)
    even = pltpu.unpack_elementwise(
        xi, index=0, packed_dtype=jnp.bfloat16,
        unpacked_dtype=jnp.float32).astype(jnp.bfloat16).reshape(h, wo, c)
    odd = pltpu.unpack_elementwise(
        xi, index=1, packed_dtype=jnp.bfloat16,
        unpacked_dtype=jnp.float32).astype(jnp.bfloat16).reshape(h, wo, c)

    lft = jnp.concatenate(
        [jnp.zeros((h, 1, c), jnp.bfloat16), odd[:, : wo - 1, :]], axis=1)
    col_taps = (lft, even, odd)

    acc = jnp.zeros((cout, m), jnp.float32)
    for kj in range(3):
        cp = col_taps[kj].reshape(ho, 2, wo, c)
        r_even = cp[:, 0]
        r_odd = cp[:, 1]
        row_taps = (
            jnp.concatenate(
                [jnp.zeros((1, wo, c), jnp.bfloat16), r_odd[: ho - 1]],
                axis=0),
            r_even,
            r_odd,
        )
        a2 = jnp.concatenate(
            [t.reshape(m, c) for t in row_taps], axis=1)
        acc += jax.lax.dot_general(
            w_ref[kj], a2, (((0,), (1,)), ((), ())),
            preferred_element_type=jnp.float32)

    return acc + b_ref[:, 0:1]


def _conv_kernel(x_ref, w_ref, b_ref, o_ref, *, h):
    o_ref[0] = _conv_one(x_ref[0].astype(jnp.bfloat16), w_ref, b_ref, h)
    o_ref[1] = _conv_one(x_ref[1].astype(jnp.bfloat16), w_ref, b_ref, h)


def kernel(x, weight, bias):
    n, c, h, w = x.shape
    cout = weight.shape[0]
    ho, wo = h // 2, w // 2
    m = ho * wo

    x2 = x.reshape(n, c, h * w)
    wt = jnp.transpose(weight, (2, 3, 1, 0)).astype(jnp.bfloat16)
    wf = jnp.transpose(wt, (1, 0, 2, 3)).reshape(3, 3 * c, cout)
    b2 = jnp.broadcast_to(bias.astype(jnp.float32)[:, None], (cout, 128))

    out = pl.pallas_call(
        functools.partial(_conv_kernel, h=h),
        out_shape=jax.ShapeDtypeStruct((n, cout, m), jnp.float32),
        grid=(n // 2,),
        in_specs=[
            pl.BlockSpec((2, c, h * w), lambda i: (i, 0, 0)),
            pl.BlockSpec((3, 3 * c, cout), lambda i: (0, 0, 0)),
            pl.BlockSpec((cout, 128), lambda i: (0, 0)),
        ],
        out_specs=pl.BlockSpec((2, cout, m), lambda i: (i, 0, 0)),
        compiler_params=pltpu.CompilerParams(
            dimension_semantics=("parallel",),
            vmem_limit_bytes=55 * 1024 * 1024),
    )(x2, wf, b2)

    return out.reshape(n, cout, ho, wo)


# speedup vs baseline: 1.0101x; 1.0101x over previous
import functools

import jax
import jax.numpy as jnp
from jax.experimental import pallas as pl
from jax.experimental.pallas import tpu as pltpu


def _conv_one(xb, w_ref, b_ref, h):
    c, hw = xb.shape
    w = hw // h
    ho, wo = h // 2, w // 2
    m = ho * wo
    cout = w_ref.shape[2]

    xt = jnp.transpose(xb)
    xi = pltpu.bitcast(xt, jnp.int32)
    even = pltpu.unpack_elementwise(
        xi, index=0, packed_dtype=jnp.bfloat16,
        unpacked_dtype=jnp.float32).astype(jnp.bfloat16).reshape(h, wo, c)
    odd = pltpu.unpack_elementwise(
        xi, index=1, packed_dtype=jnp.bfloat16,
        unpacked_dtype=jnp.float32).astype(jnp.bfloat16).reshape(h, wo, c)

    lft = jnp.concatenate(
        [jnp.zeros((h, 1, c), jnp.bfloat16), odd[:, : wo - 1, :]], axis=1)
    col_taps = (lft, even, odd)

    acc = jnp.zeros((cout, m), jnp.float32)
    for kj in range(3):
        cp = col_taps[kj].reshape(ho, 2, wo, c)
        r_even = cp[:, 0]
        r_odd = cp[:, 1]
        row_taps = (
            jnp.concatenate(
                [jnp.zeros((1, wo, c), jnp.bfloat16), r_odd[: ho - 1]],
                axis=0),
            r_even,
            r_odd,
        )
        a2 = jnp.concatenate(
            [t.reshape(m, c) for t in row_taps], axis=1)
        acc += jax.lax.dot_general(
            w_ref[kj], a2, (((0,), (1,)), ((), ())),
            preferred_element_type=jnp.float32)

    return acc + b_ref[:, 0:1]


def _conv_kernel(x_ref, w_ref, b_ref, o_ref, *, h):
    for t in range(x_ref.shape[0]):
        o_ref[t] = _conv_one(x_ref[t].astype(jnp.bfloat16), w_ref, b_ref, h)


def kernel(x, weight, bias):
    n, c, h, w = x.shape
    cout = weight.shape[0]
    ho, wo = h // 2, w // 2
    m = ho * wo

    x2 = x.reshape(n, c, h * w)
    wt = jnp.transpose(weight, (2, 3, 1, 0)).astype(jnp.bfloat16)
    wf = jnp.transpose(wt, (1, 0, 2, 3)).reshape(3, 3 * c, cout)
    b2 = jnp.broadcast_to(bias.astype(jnp.float32)[:, None], (cout, 128))

    out = pl.pallas_call(
        functools.partial(_conv_kernel, h=h),
        out_shape=jax.ShapeDtypeStruct((n, cout, m), jnp.float32),
        grid=(n // 4,),
        in_specs=[
            pl.BlockSpec((4, c, h * w), lambda i: (i, 0, 0)),
            pl.BlockSpec((3, 3 * c, cout), lambda i: (0, 0, 0)),
            pl.BlockSpec((cout, 128), lambda i: (0, 0)),
        ],
        out_specs=pl.BlockSpec((4, cout, m), lambda i: (i, 0, 0)),
        compiler_params=pltpu.CompilerParams(
            dimension_semantics=("parallel",),
            vmem_limit_bytes=55 * 1024 * 1024),
    )(x2, wf, b2)

    return out.reshape(n, cout, ho, wo)
